# baseline (speedup 1.0000x reference)
import functools

import jax
import jax.numpy as jnp
from jax import lax
from jax.experimental import pallas as pl
from jax.experimental.pallas import tpu as pltpu
from jax.experimental.pallas import tpu_sc as plsc

VOCAB = 1000000
N_WORD = 64
BATCH = 4096
SEQ = 50

_INFO = plsc.get_sparse_core_info()
_NC, _NS = _INFO.num_cores, _INFO.num_subcores
_NW = _NC * _NS
_B = BATCH * SEQ
_CHUNK = 128
_PER_W = _B // _NW
_NCHUNK = _PER_W // _CHUNK

_mesh = plsc.VectorSubcoreMesh(core_axis_name="c", subcore_axis_name="s")


@functools.partial(
    pl.kernel,
    mesh=_mesh,
    out_type=jax.ShapeDtypeStruct((_B, N_WORD), jnp.float32),
    scratch_types=[
        pltpu.VMEM((_NCHUNK, _CHUNK), jnp.int32),
        pltpu.VMEM((_CHUNK, N_WORD), jnp.float32),
        pltpu.SemaphoreType.DMA,
    ],
    compiler_params=pltpu.CompilerParams(use_tc_tiling_on_sc=False),
)
def _gather_kernel(table_hbm, idx_hbm, out_hbm, idx_v, rows_v, sem):
    wid = lax.axis_index("s") * _NC + lax.axis_index("c")
    pltpu.sync_copy(idx_hbm.at[wid], idx_v)
    base = wid * _PER_W

    def body(j, _):
        pltpu.async_copy(table_hbm.at[idx_v.at[j]], rows_v, sem).wait()
        pltpu.sync_copy(rows_v, out_hbm.at[pl.ds(base + j * _CHUNK, _CHUNK)])
        return ()

    lax.fori_loop(0, _NCHUNK, body, ())


def kernel(table, val_tok):
    idx = val_tok.reshape(_NW, _NCHUNK, _CHUNK).astype(jnp.int32)
    out = _gather_kernel(table, idx)
    return out.reshape(BATCH, SEQ, N_WORD)


# speedup vs baseline: 1.0438x; 1.0438x over previous
import functools

import jax
import jax.numpy as jnp
from jax import lax
from jax.experimental import pallas as pl
from jax.experimental.pallas import tpu as pltpu
from jax.experimental.pallas import tpu_sc as plsc

VOCAB = 1000000
N_WORD = 64
BATCH = 4096
SEQ = 50

_INFO = plsc.get_sparse_core_info()
_NC, _NS = _INFO.num_cores, _INFO.num_subcores
_NW = _NC * _NS
_B = BATCH * SEQ
_PER_W = _B // _NW
_CHUNK = 800
_NCHUNK = _PER_W // _CHUNK

_mesh = plsc.VectorSubcoreMesh(core_axis_name="c", subcore_axis_name="s")


@functools.partial(
    pl.kernel,
    mesh=_mesh,
    out_type=jax.ShapeDtypeStruct((_B, N_WORD), jnp.float32),
    scratch_types=[
        pltpu.VMEM((_PER_W,), jnp.int32),
        pltpu.VMEM((2, _CHUNK, N_WORD), jnp.float32),
        pltpu.SemaphoreType.DMA((2,)),
        pltpu.SemaphoreType.DMA((2,)),
    ],
    compiler_params=pltpu.CompilerParams(use_tc_tiling_on_sc=False),
)
def _gather_kernel(table_hbm, idx_hbm, out_hbm, idx_v, rows_v, gsem, wsem):
    rows_v = (rows_v.at[0], rows_v.at[1])
    wid = lax.axis_index("s") * _NC + lax.axis_index("c")
    pltpu.sync_copy(idx_hbm.at[wid], idx_v)
    base = wid * _PER_W

    def idx_slice(j):
        return idx_v.at[pl.ds(j * _CHUNK, _CHUNK)]

    def fire_gather(j):
        pltpu.async_copy(table_hbm.at[idx_slice(j)], rows_v[j % 2],
                         gsem.at[j % 2])

    def fire_write(j):
        pltpu.async_copy(rows_v[j % 2],
                         out_hbm.at[pl.ds(base + j * _CHUNK, _CHUNK)],
                         wsem.at[j % 2])

    fire_gather(0)
    for j in range(_NCHUNK):
        b = j % 2
        if j + 1 < _NCHUNK:
            if j >= 1:
                pltpu.make_async_copy(rows_v[(j + 1) % 2],
                                      out_hbm.at[pl.ds(0, _CHUNK)],
                                      wsem.at[(j + 1) % 2]).wait()
            fire_gather(j + 1)
        pltpu.make_async_copy(table_hbm.at[idx_slice(j)], rows_v[b],
                              gsem.at[b]).wait()
        fire_write(j)
    pltpu.make_async_copy(rows_v[(_NCHUNK - 1) % 2],
                          out_hbm.at[pl.ds(0, _CHUNK)],
                          wsem.at[(_NCHUNK - 1) % 2]).wait()


def kernel(table, val_tok):
    idx = val_tok.reshape(_NW, _PER_W).astype(jnp.int32)
    out = _gather_kernel(table, idx)
    return out.reshape(BATCH, SEQ, N_WORD)
